# baseline (speedup 1.0000x reference)
import functools

import jax
import jax.numpy as jnp
from jax import lax
from jax.experimental import pallas as pl
from jax.experimental.pallas import tpu as pltpu
from jax.experimental.pallas import tpu_sc as plsc

E = 320000
D = 128
L = 16

_info = plsc.get_sparse_core_info()
NC, NS = _info.num_cores, _info.num_subcores
NW = NC * NS
E_PER_W = E // NW
C = 80
NCHUNK = E_PER_W // C
G = C // L


def _dot_kernel(h_hbm, src_hbm, dst_hbm, out_hbm,
                src_idx, dst_idx, src_rows, dst_rows, out_all,
                sem_gs0, sem_gd0, sem_gs1, sem_gd1, sem_i0, sem_i1):
    wid = lax.axis_index("s") * NC + lax.axis_index("c")
    wbase = wid * E_PER_W
    sem_gs = (sem_gs0, sem_gs1)
    sem_gd = (sem_gd0, sem_gd1)
    sem_i = (sem_i0, sem_i1)

    def fire_idx(i, b):
        base = wbase + i * C
        pltpu.async_copy(src_hbm.at[pl.ds(base, C)], src_idx.at[b], sem_i[b])
        pltpu.async_copy(dst_hbm.at[pl.ds(base, C)], dst_idx.at[b], sem_i[b])

    def wait_idx(b):
        pltpu.make_async_copy(src_hbm.at[pl.ds(wbase, C)], src_idx.at[b],
                              sem_i[b]).wait()
        pltpu.make_async_copy(dst_hbm.at[pl.ds(wbase, C)], dst_idx.at[b],
                              sem_i[b]).wait()

    def fire_gathers(b):
        pltpu.async_copy(h_hbm.at[src_idx.at[b]], src_rows.at[b], sem_gs[b])
        pltpu.async_copy(h_hbm.at[dst_idx.at[b]], dst_rows.at[b], sem_gd[b])

    def wait_gathers(b):
        pltpu.make_async_copy(h_hbm.at[src_idx.at[b]], src_rows.at[b],
                              sem_gs[b]).wait()
        pltpu.make_async_copy(h_hbm.at[dst_idx.at[b]], dst_rows.at[b],
                              sem_gd[b]).wait()

    def compute(i, p):
        srows = src_rows.at[p]
        drows = dst_rows.at[p]

        @plsc.parallel_loop(0, G, unroll=2)
        def group_body(g):
            lane = lax.iota(jnp.int32, L)
            res = jnp.zeros((L,), jnp.float32)
            for j in range(L):
                e = g * L + j
                parts = [srows[e, pl.ds(k * L, L)] * drows[e, pl.ds(k * L, L)]
                         for k in range(D // L)]
                while len(parts) > 1:
                    parts = [a + b for a, b in zip(parts[::2], parts[1::2])]
                s = jnp.sum(parts[0])
                res = jnp.where(lane == j, s, res)
            out_all[pl.ds(i * C + g * L, L)] = res

    def step(i, p, q):
        wait_idx(q)
        fire_gathers(q)
        wait_gathers(p)
        @pl.when(i + 2 < NCHUNK)
        def _():
            fire_idx(i + 2, p)
        compute(i, p)

    fire_idx(0, 0)
    wait_idx(0)
    fire_gathers(0)
    fire_idx(1, 1)

    def pair_body(k, _):
        step(2 * k, 0, 1)
        step(2 * k + 1, 1, 0)
        return 0

    lax.fori_loop(0, (NCHUNK - 1) // 2, pair_body, 0)
    wait_gathers(0)
    compute(NCHUNK - 1, 0)

    pltpu.sync_copy(out_all, out_hbm.at[pl.ds(wbase, E_PER_W)])


@jax.jit
def kernel(h, edge_index):
    src = edge_index[0].astype(jnp.int32)
    dst = edge_index[1].astype(jnp.int32)
    mesh = plsc.VectorSubcoreMesh(core_axis_name="c", subcore_axis_name="s")
    run = pl.kernel(
        _dot_kernel,
        out_type=jax.ShapeDtypeStruct((E,), jnp.float32),
        mesh=mesh,
        compiler_params=pltpu.CompilerParams(needs_layout_passes=False),
        scratch_types=[
            pltpu.VMEM((2, C), jnp.int32),
            pltpu.VMEM((2, C), jnp.int32),
            pltpu.VMEM((2, C, D), jnp.float32),
            pltpu.VMEM((2, C, D), jnp.float32),
            pltpu.VMEM((E_PER_W,), jnp.float32),
            pltpu.SemaphoreType.DMA,
            pltpu.SemaphoreType.DMA,
            pltpu.SemaphoreType.DMA,
            pltpu.SemaphoreType.DMA,
            pltpu.SemaphoreType.DMA,
            pltpu.SemaphoreType.DMA,
        ],
    )
    return run(h, src, dst)


# speedup vs baseline: 1.2391x; 1.2391x over previous
import functools

import jax
import jax.numpy as jnp
from jax import lax
from jax.experimental import pallas as pl
from jax.experimental.pallas import tpu as pltpu
from jax.experimental.pallas import tpu_sc as plsc

E = 320000
D = 128
L = 16

_info = plsc.get_sparse_core_info()
NC, NS = _info.num_cores, _info.num_subcores
NW = NC * NS
E_PER_W = E // NW
C = 80
NCHUNK = E_PER_W // C
G = C // L


def _dot_kernel(h_hbm, src_hbm, dst_hbm, out_hbm,
                src_idx, dst_idx, src_rows, dst_rows, out_all,
                sem_gs0, sem_gd0, sem_gs1, sem_gd1, sem_i0, sem_i1):
    wid = lax.axis_index("s") * NC + lax.axis_index("c")
    wbase = wid * E_PER_W
    sem_gs = (sem_gs0, sem_gs1)
    sem_gd = (sem_gd0, sem_gd1)
    sem_i = (sem_i0, sem_i1)

    def fire_idx(i, b):
        base = wbase + i * C
        pltpu.async_copy(src_hbm.at[pl.ds(base, C)], src_idx.at[b], sem_i[b])
        pltpu.async_copy(dst_hbm.at[pl.ds(base, C)], dst_idx.at[b], sem_i[b])

    def wait_idx(b):
        pltpu.make_async_copy(src_hbm.at[pl.ds(wbase, C)], src_idx.at[b],
                              sem_i[b]).wait()
        pltpu.make_async_copy(dst_hbm.at[pl.ds(wbase, C)], dst_idx.at[b],
                              sem_i[b]).wait()

    def fire_gathers(b):
        pltpu.async_copy(h_hbm.at[src_idx.at[b]], src_rows.at[b], sem_gs[b])
        pltpu.async_copy(h_hbm.at[dst_idx.at[b]], dst_rows.at[b], sem_gd[b])

    def wait_gathers(b):
        pltpu.make_async_copy(h_hbm.at[src_idx.at[b]], src_rows.at[b],
                              sem_gs[b]).wait()
        pltpu.make_async_copy(h_hbm.at[dst_idx.at[b]], dst_rows.at[b],
                              sem_gd[b]).wait()

    def compute(i, p):
        srows = src_rows.at[p]
        drows = dst_rows.at[p]

        @plsc.parallel_loop(0, G, unroll=1)
        def group_body(g):
            base_e = i * C + g * L
            out_all[pl.ds(base_e, L)] = jnp.zeros((L,), jnp.float32)
            for j in range(L):
                e = g * L + j
                acc = srows[e, pl.ds(0, L)] * drows[e, pl.ds(0, L)]
                for k in range(1, D // L):
                    acc = acc + (srows[e, pl.ds(k * L, L)] *
                                 drows[e, pl.ds(k * L, L)])
                tgt = jnp.full((L,), base_e + j, jnp.int32)
                plsc.addupdate_scatter(out_all, [tgt], acc)

    def step(i, p, q):
        wait_idx(q)
        fire_gathers(q)
        wait_gathers(p)
        @pl.when(i + 2 < NCHUNK)
        def _():
            fire_idx(i + 2, p)
        compute(i, p)

    fire_idx(0, 0)
    wait_idx(0)
    fire_gathers(0)
    fire_idx(1, 1)

    def pair_body(k, _):
        step(2 * k, 0, 1)
        step(2 * k + 1, 1, 0)
        return 0

    lax.fori_loop(0, (NCHUNK - 1) // 2, pair_body, 0)
    wait_gathers(0)
    compute(NCHUNK - 1, 0)

    pltpu.sync_copy(out_all, out_hbm.at[pl.ds(wbase, E_PER_W)])


@jax.jit
def kernel(h, edge_index):
    src = edge_index[0].astype(jnp.int32)
    dst = edge_index[1].astype(jnp.int32)
    mesh = plsc.VectorSubcoreMesh(core_axis_name="c", subcore_axis_name="s")
    run = pl.kernel(
        _dot_kernel,
        out_type=jax.ShapeDtypeStruct((E,), jnp.float32),
        mesh=mesh,
        compiler_params=pltpu.CompilerParams(needs_layout_passes=False),
        scratch_types=[
            pltpu.VMEM((2, C), jnp.int32),
            pltpu.VMEM((2, C), jnp.int32),
            pltpu.VMEM((2, C, D), jnp.float32),
            pltpu.VMEM((2, C, D), jnp.float32),
            pltpu.VMEM((E_PER_W,), jnp.float32),
            pltpu.SemaphoreType.DMA,
            pltpu.SemaphoreType.DMA,
            pltpu.SemaphoreType.DMA,
            pltpu.SemaphoreType.DMA,
            pltpu.SemaphoreType.DMA,
            pltpu.SemaphoreType.DMA,
        ],
    )
    return run(h, src, dst)


# speedup vs baseline: 1.3682x; 1.1042x over previous
import functools

import jax
import jax.numpy as jnp
from jax import lax
from jax.experimental import pallas as pl
from jax.experimental.pallas import tpu as pltpu
from jax.experimental.pallas import tpu_sc as plsc

E = 320000
D = 128
L = 16

_info = plsc.get_sparse_core_info()
NC, NS = _info.num_cores, _info.num_subcores
NW = NC * NS
E_PER_W = E // NW
C = 80
NCHUNK = E_PER_W // C
G = C // L


def _dot_kernel(h_hbm, src_hbm, dst_hbm, out_hbm,
                src_idx, dst_idx, src_rows, dst_rows, out_all,
                sem_gs0, sem_gd0, sem_gs1, sem_gd1, sem_i0, sem_i1):
    wid = lax.axis_index("s") * NC + lax.axis_index("c")
    wbase = wid * E_PER_W
    sem_gs = (sem_gs0, sem_gs1)
    sem_gd = (sem_gd0, sem_gd1)
    sem_i = (sem_i0, sem_i1)

    def fire_idx(i, b):
        base = wbase + i * C
        pltpu.async_copy(src_hbm.at[pl.ds(base, C)], src_idx.at[b], sem_i[b])
        pltpu.async_copy(dst_hbm.at[pl.ds(base, C)], dst_idx.at[b], sem_i[b])

    def wait_idx(b):
        pltpu.make_async_copy(src_hbm.at[pl.ds(wbase, C)], src_idx.at[b],
                              sem_i[b]).wait()
        pltpu.make_async_copy(dst_hbm.at[pl.ds(wbase, C)], dst_idx.at[b],
                              sem_i[b]).wait()

    def fire_gathers(b):
        pltpu.async_copy(h_hbm.at[src_idx.at[b]], src_rows.at[b], sem_gs[b])
        pltpu.async_copy(h_hbm.at[dst_idx.at[b]], dst_rows.at[b], sem_gd[b])

    def wait_gathers(b):
        pltpu.make_async_copy(h_hbm.at[src_idx.at[b]], src_rows.at[b],
                              sem_gs[b]).wait()
        pltpu.make_async_copy(h_hbm.at[dst_idx.at[b]], dst_rows.at[b],
                              sem_gd[b]).wait()

    def compute(i, p):
        srows = src_rows.at[p]
        drows = dst_rows.at[p]

        @plsc.parallel_loop(0, G, unroll=1)
        def group_body(g):
            base_e = i * C + g * L
            out_all[pl.ds(base_e, L)] = jnp.zeros((L,), jnp.float32)
            for j in range(L):
                e = g * L + j
                acc0 = srows[e, pl.ds(0, L)] * drows[e, pl.ds(0, L)]
                acc1 = srows[e, pl.ds(L, L)] * drows[e, pl.ds(L, L)]
                for k in range(2, D // L, 2):
                    acc0 = acc0 + (srows[e, pl.ds(k * L, L)] *
                                   drows[e, pl.ds(k * L, L)])
                    acc1 = acc1 + (srows[e, pl.ds((k + 1) * L, L)] *
                                   drows[e, pl.ds((k + 1) * L, L)])
                tgt = jnp.full((L,), base_e + j, jnp.int32)
                plsc.addupdate_scatter(out_all, [tgt], acc0 + acc1)

    def step(i, p, q):
        wait_idx(q)
        fire_gathers(q)
        wait_gathers(p)
        @pl.when(i + 2 < NCHUNK)
        def _():
            fire_idx(i + 2, p)
        compute(i, p)

    fire_idx(0, 0)
    wait_idx(0)
    fire_gathers(0)
    fire_idx(1, 1)

    def pair_body(k, _):
        step(2 * k, 0, 1)
        step(2 * k + 1, 1, 0)
        return 0

    lax.fori_loop(0, (NCHUNK - 1) // 2, pair_body, 0)
    wait_gathers(0)
    compute(NCHUNK - 1, 0)

    pltpu.sync_copy(out_all, out_hbm.at[pl.ds(wbase, E_PER_W)])


@jax.jit
def kernel(h, edge_index):
    src = edge_index[0].astype(jnp.int32)
    dst = edge_index[1].astype(jnp.int32)
    mesh = plsc.VectorSubcoreMesh(core_axis_name="c", subcore_axis_name="s")
    run = pl.kernel(
        _dot_kernel,
        out_type=jax.ShapeDtypeStruct((E,), jnp.float32),
        mesh=mesh,
        compiler_params=pltpu.CompilerParams(needs_layout_passes=False),
        scratch_types=[
            pltpu.VMEM((2, C), jnp.int32),
            pltpu.VMEM((2, C), jnp.int32),
            pltpu.VMEM((2, C, D), jnp.float32),
            pltpu.VMEM((2, C, D), jnp.float32),
            pltpu.VMEM((E_PER_W,), jnp.float32),
            pltpu.SemaphoreType.DMA,
            pltpu.SemaphoreType.DMA,
            pltpu.SemaphoreType.DMA,
            pltpu.SemaphoreType.DMA,
            pltpu.SemaphoreType.DMA,
            pltpu.SemaphoreType.DMA,
        ],
    )
    return run(h, src, dst)
